# Initial kernel scaffold; baseline (speedup 1.0000x reference)
import jax
import jax.numpy as jnp
from jax.experimental import pallas as pl


def kernel(x, gate_W, gate_b, sh_w1, sh_b1, sh_w2, sh_b2, ex_w1, ex_b1, ex_w2, ex_b2):
    raise NotImplementedError("write your pallas kernel here")


# speedup vs baseline: 13.3924x; 13.3924x over previous
import functools

import jax
import jax.numpy as jnp
from jax.experimental import pallas as pl
from jax.experimental.pallas import tpu as pltpu

E = 16
TOP_K = 2
D_MODEL = 768
D_FF = 384
ALPHA = 1.702
LIMIT = 7.0

_TOK_BLK = 512


def _dot(a, b):
    return jax.lax.dot_general(a, b, (((1,), (0,)), ((), ())),
                               preferred_element_type=jnp.float32)


def _swiglu_pair(tg, tl):
    tg = jnp.clip(tg, -LIMIT, LIMIT)
    tl = jnp.clip(tl, -LIMIT, LIMIT)
    return tg * jax.nn.sigmoid(tg * ALPHA) + (tl + 1.0)


def _pad128(v):
    return jnp.concatenate([v, jnp.zeros((1, 128 - E), jnp.float32)], axis=-1)


def _top2(probs):
    iota = jax.lax.broadcasted_iota(jnp.int32, probs.shape, 1)
    p1 = jnp.max(probs, axis=-1, keepdims=True)
    i1 = jnp.min(jnp.where(probs == p1, iota, E), axis=-1, keepdims=True)
    probs_m = jnp.where(iota == i1, -1.0, probs)
    p2 = jnp.max(probs_m, axis=-1, keepdims=True)
    i2 = jnp.min(jnp.where(probs_m == p2, iota, E), axis=-1, keepdims=True)
    s = p1 + p2
    return i1, i2, p1 / s, p2 / s, iota


def _gate(x, gw_ref, gb_ref):
    logits = _dot(x, gw_ref[...]) + gb_ref[...]
    m = jnp.max(logits, axis=-1, keepdims=True)
    ex = jnp.exp(logits - m)
    probs = ex / jnp.sum(ex, axis=-1, keepdims=True)
    return logits, probs


def _experts(x, i1, i2, w1, w2, e_base, n_exp,
             exw1g_ref, exw1l_ref, exb1g_ref, exb1l_ref, exw2_ref, exb2_ref):
    y = jnp.zeros_like(x)
    for j in range(n_exp):
        e = e_base + j
        t = _swiglu_pair(_dot(x, exw1g_ref[j]) + exb1g_ref[j],
                         _dot(x, exw1l_ref[j]) + exb1l_ref[j])
        o = _dot(t, exw2_ref[j]) + exb2_ref[j]
        we = jnp.where(i1 == e, w1, 0.0) + jnp.where(i2 == e, w2, 0.0)
        y = y + o * we
    return y


def _pass1_body(num_tokens, num_blocks, n_exp,
                x_ref, gw_ref, gb_ref,
                shw1g_ref, shw1l_ref, shb1g_ref, shb1l_ref,
                shw2_ref, shb2_ref,
                exw1g_ref, exw1l_ref, exb1g_ref, exb1l_ref,
                exw2_ref, exb2_ref,
                out_ref, aux_ref, acc_ref):
    i = pl.program_id(0)

    @pl.when(i == 0)
    def _():
        acc_ref[...] = jnp.zeros_like(acc_ref)

    x = x_ref[...]
    z = _dot(_swiglu_pair(_dot(x, shw1g_ref[...]) + shb1g_ref[...],
                          _dot(x, shw1l_ref[...]) + shb1l_ref[...]),
             shw2_ref[...]) + shb2_ref[...]

    logits, probs = _gate(x, gw_ref, gb_ref)
    i1, i2, w1, w2, iota = _top2(probs)

    sel = ((iota == i1) | (iota == i2)).astype(jnp.float32)
    acc_ref[0:1, :] += _pad128(jnp.sum(probs, axis=0, keepdims=True))
    acc_ref[1:2, :] += _pad128(jnp.sum(logits, axis=0, keepdims=True))
    acc_ref[2:3, :] += _pad128(jnp.sum(sel, axis=0, keepdims=True))

    y = _experts(x, i1, i2, w1, w2, 0, n_exp,
                 exw1g_ref, exw1l_ref, exb1g_ref, exb1l_ref,
                 exw2_ref, exb2_ref)
    out_ref[...] = z + y

    @pl.when(i == num_blocks - 1)
    def _():
        P = acc_ref[0:1, 0:E] / num_tokens
        imp = acc_ref[1:2, 0:E]
        cnt = acc_ref[2:3, 0:E]
        D = cnt / (num_tokens * TOP_K)
        loss_lb = 0.01 * E * jnp.sum(P * D)
        imp_mean = jnp.sum(imp) / E
        imp_var = jnp.sum((imp - imp_mean) ** 2) / (E - 1)
        cv = jnp.sqrt(imp_var) / (imp_mean + 1e-06)
        aux = 0.01 * (loss_lb + 0.01 * cv * cv)
        aux_ref[...] = aux.reshape(1, 1)


def _pass2_body(e_base, n_exp,
                x_ref, part_ref, gw_ref, gb_ref,
                exw1g_ref, exw1l_ref, exb1g_ref, exb1l_ref,
                exw2_ref, exb2_ref,
                out_ref):
    x = x_ref[...]
    _, probs = _gate(x, gw_ref, gb_ref)
    i1, i2, w1, w2, _ = _top2(probs)
    y = _experts(x, i1, i2, w1, w2, e_base, n_exp,
                 exw1g_ref, exw1l_ref, exb1g_ref, exb1l_ref,
                 exw2_ref, exb2_ref)
    out_ref[...] = part_ref[...] + y


def kernel(x, gate_W, gate_b, sh_w1, sh_b1, sh_w2, sh_b2,
           ex_w1, ex_b1, ex_w2, ex_b2):
    shape = x.shape
    xf = x.reshape(-1, shape[-1])
    T = xf.shape[0]
    num_blocks = T // _TOK_BLK
    half = E // 2

    shw1g, shw1l = sh_w1[:, ::2], sh_w1[:, 1::2]
    shb1g, shb1l = sh_b1[::2].reshape(1, -1), sh_b1[1::2].reshape(1, -1)
    exw1g, exw1l = ex_w1[:, :, ::2], ex_w1[:, :, 1::2]
    exb1g = ex_b1[:, ::2].reshape(E, 1, D_FF)
    exb1l = ex_b1[:, 1::2].reshape(E, 1, D_FF)
    gb = gate_b.reshape(1, E)
    shb2 = sh_b2.reshape(1, D_MODEL)
    exb2 = ex_b2.reshape(E, 1, D_MODEL)

    def whole(arr):
        nd = arr.ndim
        return pl.BlockSpec(arr.shape, lambda i, _n=nd: (0,) * _n)

    tok_spec = pl.BlockSpec((_TOK_BLK, D_MODEL), lambda i: (i, 0))

    in1 = (xf, gate_W, gb, shw1g, shw1l, shb1g, shb1l, sh_w2, shb2,
           exw1g[:half], exw1l[:half], exb1g[:half], exb1l[:half],
           ex_w2[:half], exb2[:half])
    part, aux = pl.pallas_call(
        functools.partial(_pass1_body, float(T), num_blocks, half),
        grid=(num_blocks,),
        in_specs=[tok_spec] + [whole(a) for a in in1[1:]],
        out_specs=[tok_spec, pl.BlockSpec((1, 1), lambda i: (0, 0))],
        out_shape=[jax.ShapeDtypeStruct((T, D_MODEL), jnp.float32),
                   jax.ShapeDtypeStruct((1, 1), jnp.float32)],
        scratch_shapes=[pltpu.VMEM((8, 128), jnp.float32)],
    )(*in1)

    in2 = (xf, part, gate_W, gb,
           exw1g[half:], exw1l[half:], exb1g[half:], exb1l[half:],
           ex_w2[half:], exb2[half:])
    out = pl.pallas_call(
        functools.partial(_pass2_body, half, E - half),
        grid=(num_blocks,),
        in_specs=[tok_spec, tok_spec] + [whole(a) for a in in2[2:]],
        out_specs=tok_spec,
        out_shape=jax.ShapeDtypeStruct((T, D_MODEL), jnp.float32),
    )(*in2)
    return out.reshape(shape), aux[0, 0]


# speedup vs baseline: 20.2619x; 1.5129x over previous
import functools

import jax
import jax.numpy as jnp
from jax.experimental import pallas as pl
from jax.experimental.pallas import tpu as pltpu

E = 16
TOP_K = 2
D_MODEL = 768
D_FF = 384
ALPHA = 1.702
LIMIT = 7.0

_TOK_BLK = 512


def _dotf(a, b):
    return jax.lax.dot_general(a, b, (((1,), (0,)), ((), ())),
                               preferred_element_type=jnp.float32)


def _dot16(a, b):
    return jax.lax.dot_general(a.astype(jnp.bfloat16), b,
                               (((1,), (0,)), ((), ())),
                               preferred_element_type=jnp.float32)


def _swiglu_pair(tg, tl):
    tg = jnp.clip(tg, -LIMIT, LIMIT)
    tl = jnp.clip(tl, -LIMIT, LIMIT)
    return tg * jax.nn.sigmoid(tg * ALPHA) + (tl + 1.0)


def _pad128(v):
    return jnp.concatenate([v, jnp.zeros((1, 128 - E), jnp.float32)], axis=-1)


def _top2(probs):
    iota = jax.lax.broadcasted_iota(jnp.int32, probs.shape, 1)
    p1 = jnp.max(probs, axis=-1, keepdims=True)
    i1 = jnp.min(jnp.where(probs == p1, iota, E), axis=-1, keepdims=True)
    probs_m = jnp.where(iota == i1, -1.0, probs)
    p2 = jnp.max(probs_m, axis=-1, keepdims=True)
    i2 = jnp.min(jnp.where(probs_m == p2, iota, E), axis=-1, keepdims=True)
    s = p1 + p2
    return i1, i2, p1 / s, p2 / s, iota


def _body(num_tokens, num_blocks,
          x_ref, gw_ref, gb_ref,
          shw1g_ref, shw1l_ref, shb1g_ref, shb1l_ref,
          shw2_ref, shb2_ref,
          exw1g_ref, exw1l_ref, exb1g_ref, exb1l_ref,
          exw2_ref, exb2_ref,
          out_ref, aux_ref, acc_ref):
    i = pl.program_id(0)

    @pl.when(i == 0)
    def _():
        acc_ref[...] = jnp.zeros_like(acc_ref)

    x = x_ref[...]
    xh = x.astype(jnp.bfloat16)

    z = _dot16(_swiglu_pair(_dotf(xh, shw1g_ref[...]) + shb1g_ref[...],
                            _dotf(xh, shw1l_ref[...]) + shb1l_ref[...]),
               shw2_ref[...]) + shb2_ref[...]

    logits = _dotf(x, gw_ref[...]) + gb_ref[...]
    m = jnp.max(logits, axis=-1, keepdims=True)
    ex = jnp.exp(logits - m)
    probs = ex / jnp.sum(ex, axis=-1, keepdims=True)
    i1, i2, w1, w2, iota = _top2(probs)

    sel = ((iota == i1) | (iota == i2)).astype(jnp.float32)
    acc_ref[0:1, :] += _pad128(jnp.sum(probs, axis=0, keepdims=True))
    acc_ref[1:2, :] += _pad128(jnp.sum(logits, axis=0, keepdims=True))
    acc_ref[2:3, :] += _pad128(jnp.sum(sel, axis=0, keepdims=True))

    y = z
    for e in range(E):
        t = _swiglu_pair(_dotf(xh, exw1g_ref[e]) + exb1g_ref[e],
                         _dotf(xh, exw1l_ref[e]) + exb1l_ref[e])
        o = _dot16(t, exw2_ref[e]) + exb2_ref[e]
        we = jnp.where(i1 == e, w1, 0.0) + jnp.where(i2 == e, w2, 0.0)
        y = y + o * we
    out_ref[...] = y

    @pl.when(i == num_blocks - 1)
    def _():
        P = acc_ref[0:1, 0:E] / num_tokens
        imp = acc_ref[1:2, 0:E]
        cnt = acc_ref[2:3, 0:E]
        D = cnt / (num_tokens * TOP_K)
        loss_lb = 0.01 * E * jnp.sum(P * D)
        imp_mean = jnp.sum(imp) / E
        imp_var = jnp.sum((imp - imp_mean) ** 2) / (E - 1)
        cv = jnp.sqrt(imp_var) / (imp_mean + 1e-06)
        aux = 0.01 * (loss_lb + 0.01 * cv * cv)
        aux_ref[...] = aux.reshape(1, 1)


def kernel(x, gate_W, gate_b, sh_w1, sh_b1, sh_w2, sh_b2,
           ex_w1, ex_b1, ex_w2, ex_b2):
    shape = x.shape
    xf = x.reshape(-1, shape[-1])
    T = xf.shape[0]
    num_blocks = T // _TOK_BLK
    bf = jnp.bfloat16

    shw1g, shw1l = sh_w1[:, ::2].astype(bf), sh_w1[:, 1::2].astype(bf)
    shb1g, shb1l = sh_b1[::2].reshape(1, -1), sh_b1[1::2].reshape(1, -1)
    exw1g = ex_w1[:, :, ::2].astype(bf)
    exw1l = ex_w1[:, :, 1::2].astype(bf)
    exb1g = ex_b1[:, ::2].reshape(E, 1, D_FF)
    exb1l = ex_b1[:, 1::2].reshape(E, 1, D_FF)
    gb = gate_b.reshape(1, E)
    shb2 = sh_b2.reshape(1, D_MODEL)
    exb2 = ex_b2.reshape(E, 1, D_MODEL)
    shw2 = sh_w2.astype(bf)
    exw2 = ex_w2.astype(bf)

    def whole(arr):
        nd = arr.ndim
        return pl.BlockSpec(arr.shape, lambda i, _n=nd: (0,) * _n)

    tok_spec = pl.BlockSpec((_TOK_BLK, D_MODEL), lambda i: (i, 0))

    ins = (xf, gate_W, gb, shw1g, shw1l, shb1g, shb1l, shw2, shb2,
           exw1g, exw1l, exb1g, exb1l, exw2, exb2)
    out, aux = pl.pallas_call(
        functools.partial(_body, float(T), num_blocks),
        grid=(num_blocks,),
        in_specs=[tok_spec] + [whole(a) for a in ins[1:]],
        out_specs=[tok_spec, pl.BlockSpec((1, 1), lambda i: (0, 0))],
        out_shape=[jax.ShapeDtypeStruct((T, D_MODEL), jnp.float32),
                   jax.ShapeDtypeStruct((1, 1), jnp.float32)],
        scratch_shapes=[pltpu.VMEM((8, 128), jnp.float32)],
    )(*ins)
    return out.reshape(shape), aux[0, 0]


# speedup vs baseline: 23.3091x; 1.1504x over previous
import functools

import jax
import jax.numpy as jnp
from jax import lax
from jax.experimental import pallas as pl
from jax.experimental.pallas import tpu as pltpu
from jax.experimental.pallas import tpu_sc as plsc

E = 16
TOP_K = 2
D_MODEL = 768
D_FF = 384
ALPHA = 1.702
LIMIT = 7.0

GB = 512
MB = 256

_SC_CORES = 2
_SC_SUBCORES = 16
_NW = _SC_CORES * _SC_SUBCORES


def _dotf(a, b):
    return lax.dot_general(a, b, (((1,), (0,)), ((), ())),
                           preferred_element_type=jnp.float32)


def _dot16(a, b):
    return lax.dot_general(a.astype(jnp.bfloat16), b,
                           (((1,), (0,)), ((), ())),
                           preferred_element_type=jnp.float32)


def _swiglu_pair(tg, tl):
    tg = jnp.clip(tg, -LIMIT, LIMIT)
    tl = jnp.clip(tl, -LIMIT, LIMIT)
    return tg * jax.nn.sigmoid(tg * ALPHA) + (tl + 1.0)


def _pad128(v):
    return jnp.concatenate([v, jnp.zeros((1, 128 - E), jnp.float32)], axis=-1)


def _top2(probs):
    iota = lax.broadcasted_iota(jnp.int32, probs.shape, 1)
    p1 = jnp.max(probs, axis=-1, keepdims=True)
    i1 = jnp.min(jnp.where(probs == p1, iota, E), axis=-1, keepdims=True)
    probs_m = jnp.where(iota == i1, -1.0, probs)
    p2 = jnp.max(probs_m, axis=-1, keepdims=True)
    i2 = jnp.min(jnp.where(probs_m == p2, iota, E), axis=-1, keepdims=True)
    s = p1 + p2
    return i1, i2, p1 / s, p2 / s, iota


def _route_body(num_tokens, num_blocks, nb_mm,
                x_ref, gw_ref, gb_ref,
                e1_ref, e2_ref, r1_ref, r2_ref, w1_ref, w2_ref,
                segs_ref, be_ref, aux_ref, acc_ref):
    i = pl.program_id(0)

    @pl.when(i == 0)
    def _():
        acc_ref[...] = jnp.zeros_like(acc_ref)

    x = x_ref[...]
    logits = _dotf(x, gw_ref[...]) + gb_ref[...]
    m = jnp.max(logits, axis=-1, keepdims=True)
    ex = jnp.exp(logits - m)
    probs = ex / jnp.sum(ex, axis=-1, keepdims=True)
    i1, i2, w1, w2, iota = _top2(probs)

    oh1 = (iota == i1).astype(jnp.float32)
    oh2 = (iota == i2).astype(jnp.float32)

    ri = lax.broadcasted_iota(jnp.int32, (GB, GB), 0)
    ci = lax.broadcasted_iota(jnp.int32, (GB, GB), 1)
    tri = (ci < ri).astype(jnp.float32)
    rb1 = _dotf(tri, oh1)
    rb2 = _dotf(tri, oh2)
    cnt1 = jnp.sum(oh1, axis=0, keepdims=True)
    cnt2 = jnp.sum(oh2, axis=0, keepdims=True)
    base = acc_ref[2:3, 0:E]
    r1 = jnp.sum(oh1 * (rb1 + base), axis=-1, keepdims=True)
    r2 = jnp.sum(oh2 * (rb2 + base + cnt1), axis=-1, keepdims=True)

    e1_ref[...] = i1
    e2_ref[...] = i2
    r1_ref[...] = r1.astype(jnp.int32)
    r2_ref[...] = r2.astype(jnp.int32)
    w1_ref[...] = w1
    w2_ref[...] = w2

    acc_ref[0:1, :] += _pad128(jnp.sum(probs, axis=0, keepdims=True))
    acc_ref[1:2, :] += _pad128(jnp.sum(logits, axis=0, keepdims=True))
    acc_ref[2:3, :] += _pad128(cnt1 + cnt2)

    @pl.when(i == num_blocks - 1)
    def _():
        c = acc_ref[2:3, 0:E]
        nblk = jnp.ceil(c / MB)
        fi = lax.broadcasted_iota(jnp.int32, (E, E), 0)
        ei = lax.broadcasted_iota(jnp.int32, (E, E), 1)
        up = (fi < ei).astype(jnp.float32)
        segb = _dotf(nblk, up)
        segs_ref[...] = (segb * MB).astype(jnp.int32)
        rows = lax.broadcasted_iota(jnp.int32, (nb_mm, E), 0).astype(jnp.float32)
        bev = jnp.sum((segb <= rows).astype(jnp.float32), axis=-1,
                      keepdims=True) - 1.0
        be_ref[...] = bev.astype(jnp.int32)

        P = acc_ref[0:1, 0:E] / num_tokens
        imp = acc_ref[1:2, 0:E]
        D = c / (num_tokens * TOP_K)
        loss_lb = 0.01 * E * jnp.sum(P * D)
        imp_mean = jnp.sum(imp) / E
        imp_var = jnp.sum((imp - imp_mean) ** 2) / (E - 1)
        cv = jnp.sqrt(imp_var) / (imp_mean + 1e-06)
        aux = 0.01 * (loss_lb + 0.01 * cv * cv)
        aux_ref[...] = aux.reshape(1, 1)


def _route(xf, gate_W, gb, nb_mm):
    T = xf.shape[0]
    num_blocks = T // GB

    def whole(arr):
        nd = arr.ndim
        return pl.BlockSpec(arr.shape, lambda i, _n=nd: (0,) * _n)

    tok_spec = pl.BlockSpec((GB, D_MODEL), lambda i: (i, 0))
    col = lambda dt: jax.ShapeDtypeStruct((T, 1), dt)
    col_spec = pl.BlockSpec((GB, 1), lambda i: (i, 0))
    return pl.pallas_call(
        functools.partial(_route_body, float(T), num_blocks, nb_mm),
        grid=(num_blocks,),
        in_specs=[tok_spec, whole(gate_W), whole(gb)],
        out_specs=[col_spec] * 6 + [
            pl.BlockSpec((1, E), lambda i: (0, 0)),
            pl.BlockSpec((nb_mm, 1), lambda i: (0, 0)),
            pl.BlockSpec((1, 1), lambda i: (0, 0)),
        ],
        out_shape=[col(jnp.int32), col(jnp.int32), col(jnp.int32),
                   col(jnp.int32), col(jnp.float32), col(jnp.float32),
                   jax.ShapeDtypeStruct((1, E), jnp.int32),
                   jax.ShapeDtypeStruct((nb_mm, 1), jnp.int32),
                   jax.ShapeDtypeStruct((1, 1), jnp.float32)],
        scratch_shapes=[pltpu.VMEM((8, 128), jnp.float32)],
    )(xf, gate_W, gb)


def _posfix_body(e1_ref, e2_ref, r1_ref, r2_ref, segs_ref,
                 pos1_ref, pos2_ref):
    segf = segs_ref[...].astype(jnp.float32)

    def pos(e, r):
        iota = lax.broadcasted_iota(jnp.int32, (e.shape[0], E), 1)
        oh = (iota == e).astype(jnp.float32)
        return (jnp.sum(oh * segf, axis=-1, keepdims=True)
                .astype(jnp.int32) + r)

    pos1_ref[...] = pos(e1_ref[...], r1_ref[...])
    pos2_ref[...] = pos(e2_ref[...], r2_ref[...])


def _posfix(e1, e2, r1, r2, segs):
    T = e1.shape[0]
    num_blocks = T // GB
    col_spec = pl.BlockSpec((GB, 1), lambda i: (i, 0))
    return pl.pallas_call(
        _posfix_body,
        grid=(num_blocks,),
        in_specs=[col_spec] * 4 + [pl.BlockSpec((1, E), lambda i: (0, 0))],
        out_specs=[col_spec, col_spec],
        out_shape=[jax.ShapeDtypeStruct((T, 1), jnp.int32),
                   jax.ShapeDtypeStruct((T, 1), jnp.int32)],
    )(e1, e2, r1, r2, segs)


def _scatter(pos1_2d, pos2_2d, pad_n):
    nrows = pos1_2d.shape[0]
    rpt = nrows // _NW
    mesh = plsc.VectorSubcoreMesh(core_axis_name="c", subcore_axis_name="s")

    @functools.partial(
        pl.kernel,
        out_type=jax.ShapeDtypeStruct((pad_n,), jnp.int32),
        mesh=mesh,
        scratch_types=[pltpu.VMEM((rpt, 128), jnp.int32),
                       pltpu.VMEM((rpt, 128), jnp.int32),
                       pltpu.VMEM((rpt, 128), jnp.int32),
                       pltpu.SemaphoreType.DMA],
    )
    def scat(p1_h, p2_h, tok_o, i1v, i2v, tokv, sem):
        wid = lax.axis_index("s") * _SC_CORES + lax.axis_index("c")
        rowbase = wid * rpt
        pltpu.sync_copy(p1_h.at[pl.ds(rowbase, rpt)], i1v)
        pltpu.sync_copy(p2_h.at[pl.ds(rowbase, rpt)], i2v)
        for j in range(rpt):
            for q in range(8):
                tokv[j, pl.ds(q * 16, 16)] = (
                    (rowbase + j) * 128 + q * 16) + lax.iota(jnp.int32, 16)
        for j in range(rpt):
            pltpu.async_copy(tokv.at[j], tok_o.at[i1v.at[j]], sem).wait()
            pltpu.async_copy(tokv.at[j], tok_o.at[i2v.at[j]], sem).wait()

    return scat(pos1_2d, pos2_2d)


def _row_gather(table, idx, chunk_rows):
    M = idx.shape[0]
    D = table.shape[1]
    per = M // _NW
    nit = per // chunk_rows
    mesh = plsc.VectorSubcoreMesh(core_axis_name="c", subcore_axis_name="s")

    @functools.partial(
        pl.kernel,
        out_type=jax.ShapeDtypeStruct((M, D), jnp.float32),
        mesh=mesh,
        scratch_types=[pltpu.VMEM((chunk_rows,), jnp.int32),
                       pltpu.VMEM((chunk_rows, D), jnp.float32),
                       pltpu.SemaphoreType.DMA],
    )
    def gat(tab_h, idx_h, out_h, idx_v, rows_v, sem):
        wid = lax.axis_index("s") * _SC_CORES + lax.axis_index("c")
        base = wid * per
        nrows = tab_h.shape[0]
        for t in range(nit):
            off = base + t * chunk_rows
            pltpu.sync_copy(idx_h.at[pl.ds(off, chunk_rows)], idx_v)
            @pl.loop(0, chunk_rows // 16)
            def _(q):
                s = pl.ds(q * 16, 16)
                v = idx_v[s]
                idx_v[s] = jnp.minimum(jnp.maximum(v, 0), nrows - 1)
            pltpu.async_copy(tab_h.at[idx_v], rows_v, sem).wait()
            pltpu.sync_copy(rows_v, out_h.at[pl.ds(off, chunk_rows)])

    return gat(table, idx)


def _gmm_body(be_ref, xs_ref, w1g_ref, w1l_ref, b1g_ref, b1l_ref,
              w2_ref, b2_ref, ys_ref):
    del be_ref
    xh = xs_ref[...].astype(jnp.bfloat16)
    t = _swiglu_pair(_dotf(xh, w1g_ref[0]) + b1g_ref[0],
                     _dotf(xh, w1l_ref[0]) + b1l_ref[0])
    ys_ref[...] = _dot16(t, w2_ref[0]) + b2_ref[0]


def _gmm(be, xs, exw1g, exw1l, exb1g, exb1l, exw2, exb2):
    pad_n = xs.shape[0]
    nb = pad_n // MB
    grid_spec = pltpu.PrefetchScalarGridSpec(
        num_scalar_prefetch=1,
        grid=(nb,),
        in_specs=[
            pl.BlockSpec((MB, D_MODEL), lambda b, be_r: (b, 0)),
            pl.BlockSpec((1, D_MODEL, D_FF), lambda b, be_r: (be_r[b], 0, 0)),
            pl.BlockSpec((1, D_MODEL, D_FF), lambda b, be_r: (be_r[b], 0, 0)),
            pl.BlockSpec((1, 1, D_FF), lambda b, be_r: (be_r[b], 0, 0)),
            pl.BlockSpec((1, 1, D_FF), lambda b, be_r: (be_r[b], 0, 0)),
            pl.BlockSpec((1, D_FF, D_MODEL), lambda b, be_r: (be_r[b], 0, 0)),
            pl.BlockSpec((1, 1, D_MODEL), lambda b, be_r: (be_r[b], 0, 0)),
        ],
        out_specs=pl.BlockSpec((MB, D_MODEL), lambda b, be_r: (b, 0)),
    )
    return pl.pallas_call(
        _gmm_body,
        grid_spec=grid_spec,
        out_shape=jax.ShapeDtypeStruct((pad_n, D_MODEL), jnp.float32),
    )(be, xs, exw1g, exw1l, exb1g, exb1l, exw2, exb2)


def _combine_body(x_ref, y1_ref, y2_ref, w1_ref, w2_ref,
                  shw1g_ref, shw1l_ref, shb1g_ref, shb1l_ref,
                  shw2_ref, shb2_ref, out_ref):
    xh = x_ref[...].astype(jnp.bfloat16)
    z = _dot16(_swiglu_pair(_dotf(xh, shw1g_ref[...]) + shb1g_ref[...],
                            _dotf(xh, shw1l_ref[...]) + shb1l_ref[...]),
               shw2_ref[...]) + shb2_ref[...]
    out_ref[...] = (z + y1_ref[...] * w1_ref[...]
                    + y2_ref[...] * w2_ref[...])


def _combine(xf, y12, w1, w2, shw1g, shw1l, shb1g, shb1l, shw2, shb2):
    T = xf.shape[0]
    num_blocks = T // GB

    def whole(arr):
        nd = arr.ndim
        return pl.BlockSpec(arr.shape, lambda i, _n=nd: (0,) * _n)

    tok_spec = pl.BlockSpec((GB, D_MODEL), lambda i: (i, 0))
    y2_spec = pl.BlockSpec((GB, D_MODEL), lambda i, _nb=num_blocks: (i + _nb, 0))
    col_spec = pl.BlockSpec((GB, 1), lambda i: (i, 0))
    return pl.pallas_call(
        _combine_body,
        grid=(num_blocks,),
        in_specs=[tok_spec, tok_spec, y2_spec, col_spec, col_spec,
                  whole(shw1g), whole(shw1l), whole(shb1g), whole(shb1l),
                  whole(shw2), whole(shb2)],
        out_specs=tok_spec,
        out_shape=jax.ShapeDtypeStruct((T, D_MODEL), jnp.float32),
    )(xf, y12, y12, w1, w2, shw1g, shw1l, shb1g, shb1l, shw2, shb2)


def kernel(x, gate_W, gate_b, sh_w1, sh_b1, sh_w2, sh_b2,
           ex_w1, ex_b1, ex_w2, ex_b2):
    shape = x.shape
    xf = x.reshape(-1, shape[-1])
    T = xf.shape[0]
    pad_n = TOP_K * T + E * MB
    nb_mm = pad_n // MB
    bf = jnp.bfloat16

    shw1g, shw1l = sh_w1[:, ::2].astype(bf), sh_w1[:, 1::2].astype(bf)
    shb1g, shb1l = sh_b1[::2].reshape(1, -1), sh_b1[1::2].reshape(1, -1)
    exw1g = ex_w1[:, :, ::2].astype(bf)
    exw1l = ex_w1[:, :, 1::2].astype(bf)
    exb1g = ex_b1[:, ::2].reshape(E, 1, D_FF)
    exb1l = ex_b1[:, 1::2].reshape(E, 1, D_FF)
    gb = gate_b.reshape(1, E)
    shb2 = sh_b2.reshape(1, D_MODEL)
    exb2 = ex_b2.reshape(E, 1, D_MODEL)
    shw2 = sh_w2.astype(bf)
    exw2 = ex_w2.astype(bf)

    (e1, e2, r1, r2, w1, w2, segs, be, aux) = _route(xf, gate_W, gb, nb_mm)

    pos1, pos2 = _posfix(e1, e2, r1, r2, segs)
    sorted_tok = _scatter(pos1.reshape(-1, 128), pos2.reshape(-1, 128), pad_n)

    xs = _row_gather(xf, sorted_tok, 128)
    ys = _gmm(be.reshape(nb_mm), xs, exw1g, exw1l, exb1g, exb1l, exw2, exb2)
    y12 = _row_gather(
        ys, jnp.concatenate([pos1.reshape(T), pos2.reshape(T)]), 128)
    out = _combine(xf, y12, w1, w2,
                   shw1g, shw1l, shb1g, shb1l, shw2, shb2)
    return out.reshape(shape), aux[0, 0]


# speedup vs baseline: 23.4129x; 1.0045x over previous
import functools

import jax
import jax.numpy as jnp
from jax import lax
from jax.experimental import pallas as pl
from jax.experimental.pallas import tpu as pltpu
from jax.experimental.pallas import tpu_sc as plsc

E = 16
TOP_K = 2
D_MODEL = 768
D_FF = 384
ALPHA = 1.702
LIMIT = 7.0

GB = 512
MB = 256

_SC_CORES = 2
_SC_SUBCORES = 16
_NW = _SC_CORES * _SC_SUBCORES


def _dotf(a, b):
    return lax.dot_general(a, b, (((1,), (0,)), ((), ())),
                           preferred_element_type=jnp.float32)


def _dot16(a, b):
    return lax.dot_general(a.astype(jnp.bfloat16), b,
                           (((1,), (0,)), ((), ())),
                           preferred_element_type=jnp.float32)


def _swiglu_pair(tg, tl):
    tg = jnp.clip(tg, -LIMIT, LIMIT)
    tl = jnp.clip(tl, -LIMIT, LIMIT)
    return tg * jax.nn.sigmoid(tg * ALPHA) + (tl + 1.0)


def _pad128(v):
    return jnp.concatenate([v, jnp.zeros((1, 128 - E), jnp.float32)], axis=-1)


def _top2(probs):
    iota = lax.broadcasted_iota(jnp.int32, probs.shape, 1)
    p1 = jnp.max(probs, axis=-1, keepdims=True)
    i1 = jnp.min(jnp.where(probs == p1, iota, E), axis=-1, keepdims=True)
    probs_m = jnp.where(iota == i1, -1.0, probs)
    p2 = jnp.max(probs_m, axis=-1, keepdims=True)
    i2 = jnp.min(jnp.where(probs_m == p2, iota, E), axis=-1, keepdims=True)
    s = p1 + p2
    return i1, i2, p1 / s, p2 / s, iota


def _route_body(num_tokens, num_blocks, nb_mm,
                x_ref, gw_ref, gb_ref,
                e1_ref, e2_ref, r1_ref, r2_ref, w1_ref, w2_ref,
                segs_ref, be_ref, aux_ref, acc_ref):
    i = pl.program_id(0)

    @pl.when(i == 0)
    def _():
        acc_ref[...] = jnp.zeros_like(acc_ref)

    x = x_ref[...]
    logits = _dotf(x, gw_ref[...]) + gb_ref[...]
    m = jnp.max(logits, axis=-1, keepdims=True)
    ex = jnp.exp(logits - m)
    probs = ex / jnp.sum(ex, axis=-1, keepdims=True)
    i1, i2, w1, w2, iota = _top2(probs)

    oh1 = (iota == i1).astype(jnp.float32)
    oh2 = (iota == i2).astype(jnp.float32)

    ri = lax.broadcasted_iota(jnp.int32, (GB, GB), 0)
    ci = lax.broadcasted_iota(jnp.int32, (GB, GB), 1)
    tri = (ci < ri).astype(jnp.float32)
    rb1 = _dotf(tri, oh1)
    rb2 = _dotf(tri, oh2)
    cnt1 = jnp.sum(oh1, axis=0, keepdims=True)
    cnt2 = jnp.sum(oh2, axis=0, keepdims=True)
    base = acc_ref[2:3, 0:E]
    r1 = jnp.sum(oh1 * (rb1 + base), axis=-1, keepdims=True)
    r2 = jnp.sum(oh2 * (rb2 + base + cnt1), axis=-1, keepdims=True)

    e1_ref[...] = i1
    e2_ref[...] = i2
    r1_ref[...] = r1.astype(jnp.int32)
    r2_ref[...] = r2.astype(jnp.int32)
    w1_ref[...] = w1
    w2_ref[...] = w2

    acc_ref[0:1, :] += _pad128(jnp.sum(probs, axis=0, keepdims=True))
    acc_ref[1:2, :] += _pad128(jnp.sum(logits, axis=0, keepdims=True))
    acc_ref[2:3, :] += _pad128(cnt1 + cnt2)

    @pl.when(i == num_blocks - 1)
    def _():
        c = acc_ref[2:3, 0:E]
        nblk = jnp.ceil(c / MB)
        fi = lax.broadcasted_iota(jnp.int32, (E, E), 0)
        ei = lax.broadcasted_iota(jnp.int32, (E, E), 1)
        up = (fi < ei).astype(jnp.float32)
        segb = _dotf(nblk, up)
        segs_ref[...] = (segb * MB).astype(jnp.int32)
        rows = lax.broadcasted_iota(jnp.int32, (nb_mm, E), 0).astype(jnp.float32)
        bev = jnp.sum((segb <= rows).astype(jnp.float32), axis=-1,
                      keepdims=True) - 1.0
        be_ref[...] = bev.astype(jnp.int32)

        P = acc_ref[0:1, 0:E] / num_tokens
        imp = acc_ref[1:2, 0:E]
        D = c / (num_tokens * TOP_K)
        loss_lb = 0.01 * E * jnp.sum(P * D)
        imp_mean = jnp.sum(imp) / E
        imp_var = jnp.sum((imp - imp_mean) ** 2) / (E - 1)
        cv = jnp.sqrt(imp_var) / (imp_mean + 1e-06)
        aux = 0.01 * (loss_lb + 0.01 * cv * cv)
        aux_ref[...] = aux.reshape(1, 1)


def _route(xf, gate_W, gb, nb_mm):
    T = xf.shape[0]
    num_blocks = T // GB

    def whole(arr):
        nd = arr.ndim
        return pl.BlockSpec(arr.shape, lambda i, _n=nd: (0,) * _n)

    tok_spec = pl.BlockSpec((GB, D_MODEL), lambda i: (i, 0))
    col = lambda dt: jax.ShapeDtypeStruct((T, 1), dt)
    col_spec = pl.BlockSpec((GB, 1), lambda i: (i, 0))
    return pl.pallas_call(
        functools.partial(_route_body, float(T), num_blocks, nb_mm),
        grid=(num_blocks,),
        in_specs=[tok_spec, whole(gate_W), whole(gb)],
        out_specs=[col_spec] * 6 + [
            pl.BlockSpec((1, E), lambda i: (0, 0)),
            pl.BlockSpec((nb_mm, 1), lambda i: (0, 0)),
            pl.BlockSpec((1, 1), lambda i: (0, 0)),
        ],
        out_shape=[col(jnp.int32), col(jnp.int32), col(jnp.int32),
                   col(jnp.int32), col(jnp.float32), col(jnp.float32),
                   jax.ShapeDtypeStruct((1, E), jnp.int32),
                   jax.ShapeDtypeStruct((nb_mm, 1), jnp.int32),
                   jax.ShapeDtypeStruct((1, 1), jnp.float32)],
        scratch_shapes=[pltpu.VMEM((8, 128), jnp.float32)],
    )(xf, gate_W, gb)


def _posfix_body(e1_ref, e2_ref, r1_ref, r2_ref, segs_ref,
                 pos1_ref, pos2_ref):
    segf = segs_ref[...].astype(jnp.float32)

    def pos(e, r):
        iota = lax.broadcasted_iota(jnp.int32, (e.shape[0], E), 1)
        oh = (iota == e).astype(jnp.float32)
        return (jnp.sum(oh * segf, axis=-1, keepdims=True)
                .astype(jnp.int32) + r)

    pos1_ref[...] = pos(e1_ref[...], r1_ref[...])
    pos2_ref[...] = pos(e2_ref[...], r2_ref[...])


def _posfix(e1, e2, r1, r2, segs):
    T = e1.shape[0]
    num_blocks = T // GB
    col_spec = pl.BlockSpec((GB, 1), lambda i: (i, 0))
    return pl.pallas_call(
        _posfix_body,
        grid=(num_blocks,),
        in_specs=[col_spec] * 4 + [pl.BlockSpec((1, E), lambda i: (0, 0))],
        out_specs=[col_spec, col_spec],
        out_shape=[jax.ShapeDtypeStruct((T, 1), jnp.int32),
                   jax.ShapeDtypeStruct((T, 1), jnp.int32)],
    )(e1, e2, r1, r2, segs)


def _scatter(pos1_2d, pos2_2d, pad_n):
    nrows = pos1_2d.shape[0]
    rpt = nrows // _NW
    mesh = plsc.VectorSubcoreMesh(core_axis_name="c", subcore_axis_name="s")

    @functools.partial(
        pl.kernel,
        out_type=jax.ShapeDtypeStruct((pad_n,), jnp.int32),
        mesh=mesh,
        scratch_types=[pltpu.VMEM((rpt, 128), jnp.int32),
                       pltpu.VMEM((rpt, 128), jnp.int32),
                       pltpu.VMEM((rpt, 128), jnp.int32),
                       pltpu.SemaphoreType.DMA],
    )
    def scat(p1_h, p2_h, tok_o, i1v, i2v, tokv, sem):
        wid = lax.axis_index("s") * _SC_CORES + lax.axis_index("c")
        rowbase = wid * rpt
        pltpu.sync_copy(p1_h.at[pl.ds(rowbase, rpt)], i1v)
        pltpu.sync_copy(p2_h.at[pl.ds(rowbase, rpt)], i2v)
        for j in range(rpt):
            for q in range(8):
                tokv[j, pl.ds(q * 16, 16)] = (
                    (rowbase + j) * 128 + q * 16) + lax.iota(jnp.int32, 16)
        for j in range(rpt):
            pltpu.async_copy(tokv.at[j], tok_o.at[i1v.at[j]], sem).wait()
            pltpu.async_copy(tokv.at[j], tok_o.at[i2v.at[j]], sem).wait()

    return scat(pos1_2d, pos2_2d)


def _row_gather(table, idx, chunk_rows):
    M = idx.shape[0]
    D = table.shape[1]
    per = M // _NW
    nit = per // chunk_rows
    mesh = plsc.VectorSubcoreMesh(core_axis_name="c", subcore_axis_name="s")

    @functools.partial(
        pl.kernel,
        out_type=jax.ShapeDtypeStruct((M, D), jnp.float32),
        mesh=mesh,
        scratch_types=[pltpu.VMEM((chunk_rows,), jnp.int32),
                       pltpu.VMEM((chunk_rows,), jnp.int32),
                       pltpu.VMEM((chunk_rows, D), jnp.float32),
                       pltpu.VMEM((chunk_rows, D), jnp.float32),
                       pltpu.SemaphoreType.DMA,
                       pltpu.SemaphoreType.DMA],
    )
    def gat(tab_h, idx_h, out_h, idx0, idx1, rows0, rows1, sem0, sem1):
        wid = lax.axis_index("s") * _SC_CORES + lax.axis_index("c")
        base = wid * per
        nrows = tab_h.shape[0]
        idx_bufs = (idx0, idx1)
        row_bufs = (rows0, rows1)
        sems = (sem0, sem1)

        def start(t):
            iv = idx_bufs[t % 2]
            pltpu.sync_copy(idx_h.at[pl.ds(base + t * chunk_rows,
                                           chunk_rows)], iv)
            @pl.loop(0, chunk_rows // 16)
            def _(q):
                s = pl.ds(q * 16, 16)
                iv[s] = jnp.minimum(jnp.maximum(iv[s], 0), nrows - 1)
            return pltpu.async_copy(tab_h.at[iv], row_bufs[t % 2],
                                    sems[t % 2])

        dma = start(0)
        for t in range(nit):
            nxt = start(t + 1) if t + 1 < nit else None
            dma.wait()
            pltpu.sync_copy(row_bufs[t % 2],
                            out_h.at[pl.ds(base + t * chunk_rows,
                                           chunk_rows)])
            dma = nxt

    return gat(table, idx)


def _gmm_body(be_ref, xs_ref, w1g_ref, w1l_ref, b1g_ref, b1l_ref,
              w2_ref, b2_ref, ys_ref):
    del be_ref
    xh = xs_ref[...].astype(jnp.bfloat16)
    t = _swiglu_pair(_dotf(xh, w1g_ref[0]) + b1g_ref[0],
                     _dotf(xh, w1l_ref[0]) + b1l_ref[0])
    ys_ref[...] = _dot16(t, w2_ref[0]) + b2_ref[0]


def _gmm(be, xs, exw1g, exw1l, exb1g, exb1l, exw2, exb2):
    pad_n = xs.shape[0]
    nb = pad_n // MB
    grid_spec = pltpu.PrefetchScalarGridSpec(
        num_scalar_prefetch=1,
        grid=(nb,),
        in_specs=[
            pl.BlockSpec((MB, D_MODEL), lambda b, be_r: (b, 0)),
            pl.BlockSpec((1, D_MODEL, D_FF), lambda b, be_r: (be_r[b], 0, 0)),
            pl.BlockSpec((1, D_MODEL, D_FF), lambda b, be_r: (be_r[b], 0, 0)),
            pl.BlockSpec((1, 1, D_FF), lambda b, be_r: (be_r[b], 0, 0)),
            pl.BlockSpec((1, 1, D_FF), lambda b, be_r: (be_r[b], 0, 0)),
            pl.BlockSpec((1, D_FF, D_MODEL), lambda b, be_r: (be_r[b], 0, 0)),
            pl.BlockSpec((1, 1, D_MODEL), lambda b, be_r: (be_r[b], 0, 0)),
        ],
        out_specs=pl.BlockSpec((MB, D_MODEL), lambda b, be_r: (b, 0)),
    )
    return pl.pallas_call(
        _gmm_body,
        grid_spec=grid_spec,
        out_shape=jax.ShapeDtypeStruct((pad_n, D_MODEL), jnp.float32),
    )(be, xs, exw1g, exw1l, exb1g, exb1l, exw2, exb2)


def _combine_body(x_ref, y1_ref, y2_ref, w1_ref, w2_ref,
                  shw1g_ref, shw1l_ref, shb1g_ref, shb1l_ref,
                  shw2_ref, shb2_ref, out_ref):
    xh = x_ref[...].astype(jnp.bfloat16)
    z = _dot16(_swiglu_pair(_dotf(xh, shw1g_ref[...]) + shb1g_ref[...],
                            _dotf(xh, shw1l_ref[...]) + shb1l_ref[...]),
               shw2_ref[...]) + shb2_ref[...]
    out_ref[...] = (z + y1_ref[...] * w1_ref[...]
                    + y2_ref[...] * w2_ref[...])


def _combine(xf, y12, w1, w2, shw1g, shw1l, shb1g, shb1l, shw2, shb2):
    T = xf.shape[0]
    num_blocks = T // GB

    def whole(arr):
        nd = arr.ndim
        return pl.BlockSpec(arr.shape, lambda i, _n=nd: (0,) * _n)

    tok_spec = pl.BlockSpec((GB, D_MODEL), lambda i: (i, 0))
    y2_spec = pl.BlockSpec((GB, D_MODEL), lambda i, _nb=num_blocks: (i + _nb, 0))
    col_spec = pl.BlockSpec((GB, 1), lambda i: (i, 0))
    return pl.pallas_call(
        _combine_body,
        grid=(num_blocks,),
        in_specs=[tok_spec, tok_spec, y2_spec, col_spec, col_spec,
                  whole(shw1g), whole(shw1l), whole(shb1g), whole(shb1l),
                  whole(shw2), whole(shb2)],
        out_specs=tok_spec,
        out_shape=jax.ShapeDtypeStruct((T, D_MODEL), jnp.float32),
    )(xf, y12, y12, w1, w2, shw1g, shw1l, shb1g, shb1l, shw2, shb2)


def kernel(x, gate_W, gate_b, sh_w1, sh_b1, sh_w2, sh_b2,
           ex_w1, ex_b1, ex_w2, ex_b2):
    shape = x.shape
    xf = x.reshape(-1, shape[-1])
    T = xf.shape[0]
    pad_n = TOP_K * T + E * MB
    nb_mm = pad_n // MB
    bf = jnp.bfloat16

    shw1g, shw1l = sh_w1[:, ::2].astype(bf), sh_w1[:, 1::2].astype(bf)
    shb1g, shb1l = sh_b1[::2].reshape(1, -1), sh_b1[1::2].reshape(1, -1)
    exw1g = ex_w1[:, :, ::2].astype(bf)
    exw1l = ex_w1[:, :, 1::2].astype(bf)
    exb1g = ex_b1[:, ::2].reshape(E, 1, D_FF)
    exb1l = ex_b1[:, 1::2].reshape(E, 1, D_FF)
    gb = gate_b.reshape(1, E)
    shb2 = sh_b2.reshape(1, D_MODEL)
    exb2 = ex_b2.reshape(E, 1, D_MODEL)
    shw2 = sh_w2.astype(bf)
    exw2 = ex_w2.astype(bf)

    (e1, e2, r1, r2, w1, w2, segs, be, aux) = _route(xf, gate_W, gb, nb_mm)

    pos1, pos2 = _posfix(e1, e2, r1, r2, segs)
    sorted_tok = _scatter(pos1.reshape(-1, 128), pos2.reshape(-1, 128), pad_n)

    xs = _row_gather(xf, sorted_tok, 32)
    ys = _gmm(be.reshape(nb_mm), xs, exw1g, exw1l, exb1g, exb1l, exw2, exb2)
    y12 = _row_gather(
        ys, jnp.concatenate([pos1.reshape(T), pos2.reshape(T)]), 32)
    out = _combine(xf, y12, w1, w2,
                   shw1g, shw1l, shb1g, shb1l, shw2, shb2)
    return out.reshape(shape), aux[0, 0]


# speedup vs baseline: 25.1151x; 1.0727x over previous
import functools

import jax
import jax.numpy as jnp
from jax import lax
from jax.experimental import pallas as pl
from jax.experimental.pallas import tpu as pltpu
from jax.experimental.pallas import tpu_sc as plsc

E = 16
TOP_K = 2
D_MODEL = 768
D_FF = 384
ALPHA = 1.702
LIMIT = 7.0

GB = 512
MB = 256

_SC_CORES = 2
_SC_SUBCORES = 16
_NW = _SC_CORES * _SC_SUBCORES


def _dotf(a, b):
    return lax.dot_general(a, b, (((1,), (0,)), ((), ())),
                           preferred_element_type=jnp.float32)


def _dot16(a, b):
    return lax.dot_general(a.astype(jnp.bfloat16), b,
                           (((1,), (0,)), ((), ())),
                           preferred_element_type=jnp.float32)


def _swiglu_pair(tg, tl):
    tg = jnp.clip(tg, -LIMIT, LIMIT)
    tl = jnp.clip(tl, -LIMIT, LIMIT)
    return tg * jax.nn.sigmoid(tg * ALPHA) + (tl + 1.0)


def _pad128(v):
    return jnp.concatenate([v, jnp.zeros((1, 128 - E), jnp.float32)], axis=-1)


def _top2(probs):
    iota = lax.broadcasted_iota(jnp.int32, probs.shape, 1)
    p1 = jnp.max(probs, axis=-1, keepdims=True)
    i1 = jnp.min(jnp.where(probs == p1, iota, E), axis=-1, keepdims=True)
    probs_m = jnp.where(iota == i1, -1.0, probs)
    p2 = jnp.max(probs_m, axis=-1, keepdims=True)
    i2 = jnp.min(jnp.where(probs_m == p2, iota, E), axis=-1, keepdims=True)
    s = p1 + p2
    return i1, i2, p1 / s, p2 / s, iota


def _route_body(num_tokens, num_blocks, nb_mm,
                x_ref, gw_ref, gb_ref,
                e1_ref, e2_ref, r1_ref, r2_ref, w1_ref, w2_ref,
                segs_ref, be_ref, aux_ref, acc_ref):
    i = pl.program_id(0)

    @pl.when(i == 0)
    def _():
        acc_ref[...] = jnp.zeros_like(acc_ref)

    x = x_ref[...]
    logits = _dotf(x, gw_ref[...]) + gb_ref[...]
    m = jnp.max(logits, axis=-1, keepdims=True)
    ex = jnp.exp(logits - m)
    probs = ex / jnp.sum(ex, axis=-1, keepdims=True)
    i1, i2, w1, w2, iota = _top2(probs)

    oh1 = (iota == i1).astype(jnp.float32)
    oh2 = (iota == i2).astype(jnp.float32)

    ri = lax.broadcasted_iota(jnp.int32, (GB, GB), 0)
    ci = lax.broadcasted_iota(jnp.int32, (GB, GB), 1)
    tri = (ci < ri).astype(jnp.float32)
    rb1 = _dotf(tri, oh1)
    rb2 = _dotf(tri, oh2)
    cnt1 = jnp.sum(oh1, axis=0, keepdims=True)
    cnt2 = jnp.sum(oh2, axis=0, keepdims=True)
    base = acc_ref[2:3, 0:E]
    r1 = jnp.sum(oh1 * (rb1 + base), axis=-1, keepdims=True)
    r2 = jnp.sum(oh2 * (rb2 + base + cnt1), axis=-1, keepdims=True)

    e1_ref[...] = i1
    e2_ref[...] = i2
    r1_ref[...] = r1.astype(jnp.int32)
    r2_ref[...] = r2.astype(jnp.int32)
    w1_ref[...] = w1
    w2_ref[...] = w2

    acc_ref[0:1, :] += _pad128(jnp.sum(probs, axis=0, keepdims=True))
    acc_ref[1:2, :] += _pad128(jnp.sum(logits, axis=0, keepdims=True))
    acc_ref[2:3, :] += _pad128(cnt1 + cnt2)

    @pl.when(i == num_blocks - 1)
    def _():
        c = acc_ref[2:3, 0:E]
        nblk = jnp.ceil(c / MB)
        fi = lax.broadcasted_iota(jnp.int32, (E, E), 0)
        ei = lax.broadcasted_iota(jnp.int32, (E, E), 1)
        up = (fi < ei).astype(jnp.float32)
        segb = _dotf(nblk, up)
        segs_ref[...] = (segb * MB).astype(jnp.int32)
        rows = lax.broadcasted_iota(jnp.int32, (nb_mm, E), 0).astype(jnp.float32)
        bev = jnp.sum((segb <= rows).astype(jnp.float32), axis=-1,
                      keepdims=True) - 1.0
        be_ref[...] = bev.astype(jnp.int32)

        P = acc_ref[0:1, 0:E] / num_tokens
        imp = acc_ref[1:2, 0:E]
        D = c / (num_tokens * TOP_K)
        loss_lb = 0.01 * E * jnp.sum(P * D)
        imp_mean = jnp.sum(imp) / E
        imp_var = jnp.sum((imp - imp_mean) ** 2) / (E - 1)
        cv = jnp.sqrt(imp_var) / (imp_mean + 1e-06)
        aux = 0.01 * (loss_lb + 0.01 * cv * cv)
        aux_ref[...] = aux.reshape(1, 1)


def _route(xf, gate_W, gb, nb_mm):
    T = xf.shape[0]
    num_blocks = T // GB

    def whole(arr):
        nd = arr.ndim
        return pl.BlockSpec(arr.shape, lambda i, _n=nd: (0,) * _n)

    tok_spec = pl.BlockSpec((GB, D_MODEL), lambda i: (i, 0))
    col = lambda dt: jax.ShapeDtypeStruct((T, 1), dt)
    col_spec = pl.BlockSpec((GB, 1), lambda i: (i, 0))
    return pl.pallas_call(
        functools.partial(_route_body, float(T), num_blocks, nb_mm),
        grid=(num_blocks,),
        in_specs=[tok_spec, whole(gate_W), whole(gb)],
        out_specs=[col_spec] * 6 + [
            pl.BlockSpec((1, E), lambda i: (0, 0)),
            pl.BlockSpec((nb_mm, 1), lambda i: (0, 0)),
            pl.BlockSpec((1, 1), lambda i: (0, 0)),
        ],
        out_shape=[col(jnp.int32), col(jnp.int32), col(jnp.int32),
                   col(jnp.int32), col(jnp.float32), col(jnp.float32),
                   jax.ShapeDtypeStruct((1, E), jnp.int32),
                   jax.ShapeDtypeStruct((nb_mm, 1), jnp.int32),
                   jax.ShapeDtypeStruct((1, 1), jnp.float32)],
        scratch_shapes=[pltpu.VMEM((8, 128), jnp.float32)],
    )(xf, gate_W, gb)


def _posfix_body(e1_ref, e2_ref, r1_ref, r2_ref, segs_ref,
                 pos1_ref, pos2_ref):
    segf = segs_ref[...].astype(jnp.float32)

    def pos(e, r):
        iota = lax.broadcasted_iota(jnp.int32, (e.shape[0], E), 1)
        oh = (iota == e).astype(jnp.float32)
        return (jnp.sum(oh * segf, axis=-1, keepdims=True)
                .astype(jnp.int32) + r)

    pos1_ref[...] = pos(e1_ref[...], r1_ref[...])
    pos2_ref[...] = pos(e2_ref[...], r2_ref[...])


def _posfix(e1, e2, r1, r2, segs):
    T = e1.shape[0]
    num_blocks = T // GB
    col_spec = pl.BlockSpec((GB, 1), lambda i: (i, 0))
    return pl.pallas_call(
        _posfix_body,
        grid=(num_blocks,),
        in_specs=[col_spec] * 4 + [pl.BlockSpec((1, E), lambda i: (0, 0))],
        out_specs=[col_spec, col_spec],
        out_shape=[jax.ShapeDtypeStruct((T, 1), jnp.int32),
                   jax.ShapeDtypeStruct((T, 1), jnp.int32)],
    )(e1, e2, r1, r2, segs)


def _dispatch(xf, pos1_2d, pos2_2d, pad_n):
    D = xf.shape[1]
    nrows = pos1_2d.shape[0]
    rpt = nrows // _NW
    mesh = plsc.VectorSubcoreMesh(core_axis_name="c", subcore_axis_name="s")

    @functools.partial(
        pl.kernel,
        out_type=jax.ShapeDtypeStruct((pad_n, D), jnp.float32),
        mesh=mesh,
        scratch_types=[pltpu.VMEM((rpt, 128), jnp.int32),
                       pltpu.VMEM((rpt, 128), jnp.int32),
                       pltpu.VMEM((128, D), jnp.float32),
                       pltpu.SemaphoreType.DMA],
    )
    def disp(x_h, p1_h, p2_h, xs_o, i1v, i2v, rows_v, sem):
        wid = lax.axis_index("s") * _SC_CORES + lax.axis_index("c")
        rowbase = wid * rpt
        pltpu.sync_copy(p1_h.at[pl.ds(rowbase, rpt)], i1v)
        pltpu.sync_copy(p2_h.at[pl.ds(rowbase, rpt)], i2v)
        for j in range(rpt):
            pltpu.sync_copy(x_h.at[pl.ds((rowbase + j) * 128, 128)], rows_v)
            pltpu.async_copy(rows_v, xs_o.at[i1v.at[j]], sem).wait()
            pltpu.async_copy(rows_v, xs_o.at[i2v.at[j]], sem).wait()

    return disp(xf, pos1_2d, pos2_2d)


def _row_gather(table, idx, chunk_rows):
    M = idx.shape[0]
    D = table.shape[1]
    per = M // _NW
    nit = per // chunk_rows
    mesh = plsc.VectorSubcoreMesh(core_axis_name="c", subcore_axis_name="s")

    @functools.partial(
        pl.kernel,
        out_type=jax.ShapeDtypeStruct((M, D), jnp.float32),
        mesh=mesh,
        scratch_types=[pltpu.VMEM((chunk_rows,), jnp.int32),
                       pltpu.VMEM((chunk_rows,), jnp.int32),
                       pltpu.VMEM((chunk_rows, D), jnp.float32),
                       pltpu.VMEM((chunk_rows, D), jnp.float32),
                       pltpu.SemaphoreType.DMA,
                       pltpu.SemaphoreType.DMA],
    )
    def gat(tab_h, idx_h, out_h, idx0, idx1, rows0, rows1, sem0, sem1):
        wid = lax.axis_index("s") * _SC_CORES + lax.axis_index("c")
        base = wid * per
        nrows = tab_h.shape[0]
        idx_bufs = (idx0, idx1)
        row_bufs = (rows0, rows1)
        sems = (sem0, sem1)

        def start(t):
            iv = idx_bufs[t % 2]
            pltpu.sync_copy(idx_h.at[pl.ds(base + t * chunk_rows,
                                           chunk_rows)], iv)
            @pl.loop(0, chunk_rows // 16)
            def _(q):
                s = pl.ds(q * 16, 16)
                iv[s] = jnp.minimum(jnp.maximum(iv[s], 0), nrows - 1)
            return pltpu.async_copy(tab_h.at[iv], row_bufs[t % 2],
                                    sems[t % 2])

        dma = start(0)
        for t in range(nit):
            nxt = start(t + 1) if t + 1 < nit else None
            dma.wait()
            pltpu.sync_copy(row_bufs[t % 2],
                            out_h.at[pl.ds(base + t * chunk_rows,
                                           chunk_rows)])
            dma = nxt

    return gat(table, idx)


def _gmm_body(be_ref, xs_ref, w1g_ref, w1l_ref, b1g_ref, b1l_ref,
              w2_ref, b2_ref, ys_ref):
    del be_ref
    xh = xs_ref[...].astype(jnp.bfloat16)
    t = _swiglu_pair(_dotf(xh, w1g_ref[0]) + b1g_ref[0],
                     _dotf(xh, w1l_ref[0]) + b1l_ref[0])
    ys_ref[...] = _dot16(t, w2_ref[0]) + b2_ref[0]


def _gmm(be, xs, exw1g, exw1l, exb1g, exb1l, exw2, exb2):
    pad_n = xs.shape[0]
    nb = pad_n // MB
    grid_spec = pltpu.PrefetchScalarGridSpec(
        num_scalar_prefetch=1,
        grid=(nb,),
        in_specs=[
            pl.BlockSpec((MB, D_MODEL), lambda b, be_r: (b, 0)),
            pl.BlockSpec((1, D_MODEL, D_FF), lambda b, be_r: (be_r[b], 0, 0)),
            pl.BlockSpec((1, D_MODEL, D_FF), lambda b, be_r: (be_r[b], 0, 0)),
            pl.BlockSpec((1, 1, D_FF), lambda b, be_r: (be_r[b], 0, 0)),
            pl.BlockSpec((1, 1, D_FF), lambda b, be_r: (be_r[b], 0, 0)),
            pl.BlockSpec((1, D_FF, D_MODEL), lambda b, be_r: (be_r[b], 0, 0)),
            pl.BlockSpec((1, 1, D_MODEL), lambda b, be_r: (be_r[b], 0, 0)),
        ],
        out_specs=pl.BlockSpec((MB, D_MODEL), lambda b, be_r: (b, 0)),
    )
    return pl.pallas_call(
        _gmm_body,
        grid_spec=grid_spec,
        out_shape=jax.ShapeDtypeStruct((pad_n, D_MODEL), jnp.float32),
    )(be, xs, exw1g, exw1l, exb1g, exb1l, exw2, exb2)


def _combine_body(x_ref, y1_ref, y2_ref, w1_ref, w2_ref,
                  shw1g_ref, shw1l_ref, shb1g_ref, shb1l_ref,
                  shw2_ref, shb2_ref, out_ref):
    xh = x_ref[...].astype(jnp.bfloat16)
    z = _dot16(_swiglu_pair(_dotf(xh, shw1g_ref[...]) + shb1g_ref[...],
                            _dotf(xh, shw1l_ref[...]) + shb1l_ref[...]),
               shw2_ref[...]) + shb2_ref[...]
    out_ref[...] = (z + y1_ref[...] * w1_ref[...]
                    + y2_ref[...] * w2_ref[...])


def _combine(xf, y12, w1, w2, shw1g, shw1l, shb1g, shb1l, shw2, shb2):
    T = xf.shape[0]
    num_blocks = T // GB

    def whole(arr):
        nd = arr.ndim
        return pl.BlockSpec(arr.shape, lambda i, _n=nd: (0,) * _n)

    tok_spec = pl.BlockSpec((GB, D_MODEL), lambda i: (i, 0))
    y2_spec = pl.BlockSpec((GB, D_MODEL), lambda i, _nb=num_blocks: (i + _nb, 0))
    col_spec = pl.BlockSpec((GB, 1), lambda i: (i, 0))
    return pl.pallas_call(
        _combine_body,
        grid=(num_blocks,),
        in_specs=[tok_spec, tok_spec, y2_spec, col_spec, col_spec,
                  whole(shw1g), whole(shw1l), whole(shb1g), whole(shb1l),
                  whole(shw2), whole(shb2)],
        out_specs=tok_spec,
        out_shape=jax.ShapeDtypeStruct((T, D_MODEL), jnp.float32),
    )(xf, y12, y12, w1, w2, shw1g, shw1l, shb1g, shb1l, shw2, shb2)


def kernel(x, gate_W, gate_b, sh_w1, sh_b1, sh_w2, sh_b2,
           ex_w1, ex_b1, ex_w2, ex_b2):
    shape = x.shape
    xf = x.reshape(-1, shape[-1])
    T = xf.shape[0]
    pad_n = TOP_K * T + E * MB
    nb_mm = pad_n // MB
    bf = jnp.bfloat16

    shw1g, shw1l = sh_w1[:, ::2].astype(bf), sh_w1[:, 1::2].astype(bf)
    shb1g, shb1l = sh_b1[::2].reshape(1, -1), sh_b1[1::2].reshape(1, -1)
    exw1g = ex_w1[:, :, ::2].astype(bf)
    exw1l = ex_w1[:, :, 1::2].astype(bf)
    exb1g = ex_b1[:, ::2].reshape(E, 1, D_FF)
    exb1l = ex_b1[:, 1::2].reshape(E, 1, D_FF)
    gb = gate_b.reshape(1, E)
    shb2 = sh_b2.reshape(1, D_MODEL)
    exb2 = ex_b2.reshape(E, 1, D_MODEL)
    shw2 = sh_w2.astype(bf)
    exw2 = ex_w2.astype(bf)

    (e1, e2, r1, r2, w1, w2, segs, be, aux) = _route(xf, gate_W, gb, nb_mm)

    pos1, pos2 = _posfix(e1, e2, r1, r2, segs)
    xs = _dispatch(xf, pos1.reshape(-1, 128), pos2.reshape(-1, 128), pad_n)
    ys = _gmm(be.reshape(nb_mm), xs, exw1g, exw1l, exb1g, exb1l, exw2, exb2)
    y12 = _row_gather(
        ys, jnp.concatenate([pos1.reshape(T), pos2.reshape(T)]), 32)
    out = _combine(xf, y12, w1, w2,
                   shw1g, shw1l, shb1g, shb1l, shw2, shb2)
    return out.reshape(shape), aux[0, 0]
